# Initial kernel scaffold; baseline (speedup 1.0000x reference)
import jax
import jax.numpy as jnp
from jax.experimental import pallas as pl


def kernel(x, phi, pos_table):
    raise NotImplementedError("write your pallas kernel here")


# speedup vs baseline: 2.4172x; 2.4172x over previous
import functools

import jax
import jax.numpy as jnp
from jax import lax
from jax.experimental import pallas as pl

_BLK = 64
_TOPK = 8


def _attn_block_kernel(x_ref, phi_ref, pos_ref, o_ref, *, batch, d):
    vrow = phi_ref[...].reshape(1, d)
    vcol = vrow.reshape(d, 1)
    ii = lax.broadcasted_iota(jnp.int32, (d, d), 0)
    kk = lax.broadcasted_iota(jnp.int32, (d, d), 1)
    vi = jnp.broadcast_to(vcol, (d, d))
    vk = jnp.broadcast_to(vrow, (d, d))
    tie_before = (vk == vi) & (kk < ii)
    rank_desc = jnp.sum(((vk > vi) | tie_before).astype(jnp.float32),
                        axis=1, keepdims=True)
    rank_asc = jnp.sum(((vk < vi) | tie_before).astype(jnp.float32),
                       axis=1, keepdims=True)
    hp = jnp.max((vrow > 0).astype(jnp.float32), axis=(0, 1), keepdims=True)
    hn = jnp.max((vrow < 0).astype(jnp.float32), axis=(0, 1), keepdims=True)
    hz = jnp.max((vrow == 0).astype(jnp.float32), axis=(0, 1), keepdims=True)
    i_col = lax.broadcasted_iota(jnp.int32, (d, 1), 0)
    sel = (((rank_desc < _TOPK) & (hp > 0))
           | ((rank_asc < _TOPK) & (hn > 0))
           | ((i_col < _TOPK) & (hz > 0)))
    sel_f = sel.astype(jnp.float32)

    pos_b = pos_ref[...]
    for b in range(batch):
        xb = x_ref[b]
        q = xb + pos_b + vrow
        sc = jnp.dot(q, q.T, preferred_element_type=jnp.float32) * 0.125
        m = jnp.max(sc, axis=1, keepdims=True)
        e = jnp.exp(sc - m)
        p = e / jnp.sum(e, axis=1, keepdims=True)
        p = p * sel_f
        o_ref[b] = jnp.dot(p, xb, preferred_element_type=jnp.float32)


@jax.jit
def kernel(x, phi, pos_table):
    batch, s, d = x.shape
    nb = s // _BLK
    body = functools.partial(_attn_block_kernel, batch=batch, d=d)
    return pl.pallas_call(
        body,
        grid=(nb,),
        in_specs=[
            pl.BlockSpec((batch, _BLK, d), lambda i: (0, i, 0)),
            pl.BlockSpec((1, 1, d), lambda i: (0, 0, 0)),
            pl.BlockSpec((_BLK, d), lambda i: (i, 0)),
        ],
        out_specs=pl.BlockSpec((batch, _BLK, d), lambda i: (0, i, 0)),
        out_shape=jax.ShapeDtypeStruct((batch, s, d), jnp.float32),
    )(x, phi, pos_table)


# speedup vs baseline: 5.4331x; 2.2477x over previous
import functools

import jax
import jax.numpy as jnp
from jax import lax
from jax.experimental import pallas as pl
from jax.experimental.pallas import tpu as pltpu

_BLK = 64
_TOPK = 8
_GROUP = 3


def _row_select(phi_ref, d):
    vrow = phi_ref[...].reshape(1, d)
    vcol = vrow.reshape(d, 1)
    ii = lax.broadcasted_iota(jnp.int32, (d, d), 0)
    kk = lax.broadcasted_iota(jnp.int32, (d, d), 1)
    vi = jnp.broadcast_to(vcol, (d, d))
    vk = jnp.broadcast_to(vrow, (d, d))
    tie_before = (vk == vi) & (kk < ii)
    rank_desc = jnp.sum(((vk > vi) | tie_before).astype(jnp.float32),
                        axis=1, keepdims=True)
    rank_asc = jnp.sum(((vk < vi) | tie_before).astype(jnp.float32),
                       axis=1, keepdims=True)
    hp = jnp.max((vrow > 0).astype(jnp.float32), axis=(0, 1), keepdims=True)
    hn = jnp.max((vrow < 0).astype(jnp.float32), axis=(0, 1), keepdims=True)
    hz = jnp.max((vrow == 0).astype(jnp.float32), axis=(0, 1), keepdims=True)
    i_col = lax.broadcasted_iota(jnp.int32, (d, 1), 0)
    sel = (((rank_desc < _TOPK) & (hp > 0))
           | ((rank_asc < _TOPK) & (hn > 0))
           | ((i_col < _TOPK) & (hz > 0)))
    return sel.astype(jnp.float32)


def _attn_kernel(x_ref, phi_ref, pos_ref, o_ref, mask_ref, selc_ref,
                 *, batch, d):
    rows = batch * _BLK

    @pl.when(pl.program_id(0) == 0)
    def _init():
        sel = _row_select(phi_ref, d)
        sel4 = jnp.broadcast_to(sel.reshape(1, _BLK, 1),
                                (batch, _BLK, 1)).reshape(rows, 1)
        selc_ref[...] = sel4
        ii = lax.broadcasted_iota(jnp.int32, (rows, rows), 0)
        jj = lax.broadcasted_iota(jnp.int32, (rows, rows), 1)
        same = (ii // _BLK) == (jj // _BLK)
        mask_ref[...] = jnp.where(same, 0.0, -1e30)

    vrow = phi_ref[...].reshape(1, d)
    amask = mask_ref[...]
    selc = selc_ref[...]
    for k in range(_GROUP):
        xb = x_ref[:, k * _BLK:(k + 1) * _BLK, :].reshape(rows, d)
        posb = pos_ref[k * _BLK:(k + 1) * _BLK, :]
        posq = jnp.broadcast_to(posb.reshape(1, _BLK, d),
                                (batch, _BLK, d)).reshape(rows, d)
        q = xb + posq + vrow
        s = jnp.dot(q, q.T, preferred_element_type=jnp.float32) * 0.125
        s = s + amask
        m = jnp.max(s, axis=1, keepdims=True)
        e = jnp.exp(s - m)
        r = jnp.sum(e, axis=1, keepdims=True)
        p = e * (selc / r)
        o = jnp.dot(p, xb, preferred_element_type=jnp.float32)
        o_ref[:, k * _BLK:(k + 1) * _BLK, :] = o.reshape(batch, _BLK, d)


@jax.jit
def kernel(x, phi, pos_table):
    batch, s, d = x.shape
    ng = s // (_BLK * _GROUP)
    rows = batch * _BLK
    body = functools.partial(_attn_kernel, batch=batch, d=d)
    return pl.pallas_call(
        body,
        grid=(ng,),
        in_specs=[
            pl.BlockSpec((batch, _BLK * _GROUP, d), lambda i: (0, i, 0)),
            pl.BlockSpec((1, 1, d), lambda i: (0, 0, 0)),
            pl.BlockSpec((_BLK * _GROUP, d), lambda i: (i, 0)),
        ],
        out_specs=pl.BlockSpec((batch, _BLK * _GROUP, d), lambda i: (0, i, 0)),
        out_shape=jax.ShapeDtypeStruct((batch, s, d), jnp.float32),
        scratch_shapes=[
            pltpu.VMEM((rows, rows), jnp.float32),
            pltpu.VMEM((rows, 1), jnp.float32),
        ],
    )(x, phi, pos_table)
